# baseline (speedup 1.0000x reference)
import jax
import jax.numpy as jnp
import numpy as np
from jax import lax
from jax.experimental import pallas as pl
from jax.experimental.pallas import tpu as pltpu

_N_ANCHORS = 9
_FEAT_STRIDE = 16
_PRE_NMS = 6000
_POST_NMS = 300
_NMS_THRESH = 0.7
_MIN_SIZE = 16
_PAD = 6016
_ROWS = _PAD // 128


def _whctrs(a):
    w = a[2] - a[0] + 1.0
    h = a[3] - a[1] + 1.0
    return w, h, a[0] + 0.5 * (w - 1), a[1] + 0.5 * (h - 1)


def _mkanchors(ws, hs, x, y):
    ws = ws[:, None]
    hs = hs[:, None]
    return np.hstack((x - 0.5 * (ws - 1), y - 0.5 * (hs - 1),
                      x + 0.5 * (ws - 1), y + 0.5 * (hs - 1)))


def _gen_anchors(base_size=16, ratios=(0.5, 1.0, 2.0), scales=(8, 16, 32)):
    base = np.array([1, 1, base_size, base_size], dtype=np.float64) - 1
    w, h, x, y = _whctrs(base)
    size = w * h
    ws = np.round(np.sqrt(size / np.array(ratios)))
    hs = np.round(ws * np.array(ratios))
    ra = _mkanchors(ws, hs, x, y)
    out = []
    for i in range(ra.shape[0]):
        w, h, x, y = _whctrs(ra[i])
        out.append(_mkanchors(w * np.array(scales), h * np.array(scales), x, y))
    return np.vstack(out).astype(np.float32)


def _conv(x, w, b, pad):
    y = lax.conv_general_dilated(x, w, (1, 1), [(pad, pad), (pad, pad)],
                                 dimension_numbers=('NCHW', 'OIHW', 'NCHW'))
    return y + b[None, :, None, None]


def _nms_body(x1_ref, y1_ref, x2_ref, y2_ref, out_ref, aidx_scr):
    x1 = x1_ref[...]
    y1 = y1_ref[...]
    x2 = x2_ref[...]
    y2 = y2_ref[...]
    areas = (x2 - x1 + 1.0) * (y2 - y1 + 1.0)
    ridx = lax.broadcasted_iota(jnp.int32, (_ROWS, 128), 0)
    cidx = lax.broadcasted_iota(jnp.int32, (_ROWS, 128), 1)
    fidx = ridx * 128 + cidx
    big = jnp.int32(2**30)
    aidx_scr[...] = jnp.where(fidx < _PRE_NMS, fidx, big)
    lane1 = lax.broadcasted_iota(jnp.int32, (1, 128), 1)
    neg = jnp.float32(-jnp.inf)

    def body(i, carry):
        aidx = aidx_scr[...]
        jraw = jnp.min(aidx)
        j = jnp.where(jraw >= big, 0, jraw)
        r = j // 128
        c = j - r * 128
        pickl = lane1 == c
        x1j = jnp.max(jnp.where(pickl, x1_ref[pl.ds(r, 1), :], neg))
        y1j = jnp.max(jnp.where(pickl, y1_ref[pl.ds(r, 1), :], neg))
        x2j = jnp.max(jnp.where(pickl, x2_ref[pl.ds(r, 1), :], neg))
        y2j = jnp.max(jnp.where(pickl, y2_ref[pl.ds(r, 1), :], neg))
        aj = (x2j - x1j + 1.0) * (y2j - y1j + 1.0)
        xx1 = jnp.maximum(x1j, x1)
        yy1 = jnp.maximum(y1j, y1)
        xx2 = jnp.minimum(x2j, x2)
        yy2 = jnp.minimum(y2j, y2)
        iw = jnp.maximum(0.0, xx2 - xx1 + 1.0)
        ih = jnp.maximum(0.0, yy2 - yy1 + 1.0)
        inter = iw * ih
        ovr = inter / (aj + areas - inter)
        aidx_scr[...] = jnp.where(ovr > _NMS_THRESH, big, aidx)
        row = jnp.where(lane1 == 1, x1j,
              jnp.where(lane1 == 2, y1j,
              jnp.where(lane1 == 3, x2j,
              jnp.where(lane1 == 4, y2j, 0.0))))
        out_ref[pl.ds(i, 1), :] = row
        return carry

    lax.fori_loop(0, _POST_NMS, body, 0)


def _nms_pallas(props):
    npad = _PAD - _PRE_NMS
    x1 = jnp.concatenate([props[:, 0], jnp.zeros((npad,), jnp.float32)])
    y1 = jnp.concatenate([props[:, 1], jnp.zeros((npad,), jnp.float32)])
    x2 = jnp.concatenate([props[:, 2], jnp.full((npad,), -1.0, jnp.float32)])
    y2 = jnp.concatenate([props[:, 3], jnp.full((npad,), -1.0, jnp.float32)])
    args = [a.reshape(_ROWS, 128) for a in (x1, y1, x2, y2)]
    out = pl.pallas_call(
        _nms_body,
        out_shape=jax.ShapeDtypeStruct((_POST_NMS, 128), jnp.float32),
        scratch_shapes=[pltpu.VMEM((_ROWS, 128), jnp.int32)],
    )(*args)
    return out[:, :5]


def kernel(features, gt_boxes, im_info, conv_w, conv_b, cls_w, cls_b, box_w, box_b):
    bsize, _, h, w = features.shape
    rpn_feat = jax.nn.relu(_conv(features, conv_w, conv_b, 1))
    cls_score = _conv(rpn_feat, cls_w, cls_b, 0)
    cls_prob = jax.nn.softmax(cls_score.reshape(bsize, 2, _N_ANCHORS, h, w),
                              axis=1).reshape(bsize, 2 * _N_ANCHORS, h, w)
    box_reg = _conv(rpn_feat, box_w, box_b, 0)

    scores = cls_prob[:, _N_ANCHORS:, :, :].transpose(0, 2, 3, 1).reshape(-1)
    deltas = box_reg.transpose(0, 2, 3, 1).reshape(-1, 4)

    anchors = jnp.asarray(_gen_anchors(_FEAT_STRIDE))
    sx = np.arange(w) * _FEAT_STRIDE
    sy = np.arange(h) * _FEAT_STRIDE
    sx, sy = np.meshgrid(sx, sy)
    shifts = jnp.asarray(np.stack([sx.ravel(), sy.ravel(), sx.ravel(), sy.ravel()],
                                  axis=1).astype(np.float32))
    all_anchors = (anchors[None, :, :] + shifts[:, None, :]).reshape(-1, 4)

    ws = all_anchors[:, 2] - all_anchors[:, 0] + 1.0
    hs = all_anchors[:, 3] - all_anchors[:, 1] + 1.0
    cx = all_anchors[:, 0] + 0.5 * ws
    cy = all_anchors[:, 1] + 0.5 * hs
    dx, dy, dw, dh = deltas[:, 0], deltas[:, 1], deltas[:, 2], deltas[:, 3]
    pcx = dx * ws + cx
    pcy = dy * hs + cy
    pw = jnp.exp(dw) * ws
    ph = jnp.exp(dh) * hs
    proposals = jnp.stack([pcx - 0.5 * pw, pcy - 0.5 * ph,
                           pcx + 0.5 * pw, pcy + 0.5 * ph], axis=1)
    H = im_info[0, 0]
    W = im_info[0, 1]
    proposals = jnp.stack([
        jnp.clip(proposals[:, 0], 0.0, W - 1.0),
        jnp.clip(proposals[:, 1], 0.0, H - 1.0),
        jnp.clip(proposals[:, 2], 0.0, W - 1.0),
        jnp.clip(proposals[:, 3], 0.0, H - 1.0)], axis=1)

    min_size = _MIN_SIZE * im_info[0, 2]
    pws = proposals[:, 2] - proposals[:, 0] + 1.0
    phs = proposals[:, 3] - proposals[:, 1] + 1.0
    valid = (pws >= min_size) & (phs >= min_size)
    scores = jnp.where(valid, scores, -1e9)

    top_scores, order = lax.top_k(scores, _PRE_NMS)
    props = proposals[order]
    return _nms_pallas(props)
